# baseline (speedup 1.0000x reference)
import jax
import jax.numpy as jnp
from jax.experimental import pallas as pl

S = 8192
K = 64
BC = 256
G0 = S // K


def _rev1(x):
    N, L, M, C = x.shape
    d = L // 2
    while d >= 1:
        y = x.reshape(-1, 2, d, M, C)
        x = jnp.concatenate([y[:, 1], y[:, 0]], axis=1).reshape(N, L, M, C)
        d //= 2
    return x


def _merge_down1(x):
    N, L, M, C = x.shape
    d = L // 2
    while d >= 1:
        y = x.reshape(-1, 2, d, M, C)
        a, b = y[:, 0], y[:, 1]
        x = jnp.concatenate(
            [jnp.maximum(a, b), jnp.minimum(a, b)], axis=1
        ).reshape(N, L, M, C)
        d //= 2
    return x


def _topk_kernel(x_ref, o_ref):
    x = x_ref[0].reshape(K, 1, G0, BC)
    L = 1
    while L < K:
        y = x.reshape(-1, 2, L, G0, BC)
        a, b = y[:, 0], y[:, 1]
        x = _merge_down1(jnp.concatenate([a, _rev1(b)], axis=1))
        L *= 2
    cur = x
    g = G0
    while g > 1:
        h = g // 2
        a, b = cur[:, :, :h, :], cur[:, :, h:, :]
        m = jnp.maximum(a, _rev1(b))
        cur = _merge_down1(m)
        g = h
    o_ref[0] = cur[0, :, 0, :]


def kernel(inputs):
    B, s, C = inputs.shape
    assert s == S and C % BC == 0
    grid = (B, C // BC)
    return pl.pallas_call(
        _topk_kernel,
        grid=grid,
        in_specs=[pl.BlockSpec((1, S, BC), lambda b, c: (b, 0, c))],
        out_specs=pl.BlockSpec((1, K, BC), lambda b, c: (b, 0, c)),
        out_shape=jax.ShapeDtypeStruct((B, K, C), jnp.float32),
    )(inputs)


# speedup vs baseline: 2.0596x; 2.0596x over previous
import jax
import jax.numpy as jnp
from jax.experimental import pallas as pl

S = 8192
K = 64
BC = 128
G0 = S // K


def _merge(pieces, desc):
    n = len(pieces)
    if n == 1:
        return pieces
    h = n // 2
    hi, lo = [], []
    for i in range(h):
        hi.append(jnp.maximum(pieces[i], pieces[i + h]))
        lo.append(jnp.minimum(pieces[i], pieces[i + h]))
    if desc:
        return _merge(hi, True) + _merge(lo, True)
    return _merge(lo, False) + _merge(hi, False)


def _sort(pieces, desc):
    n = len(pieces)
    if n == 1:
        return pieces
    h = n // 2
    a = _sort(pieces[:h], True)
    b = _sort(pieces[h:], False)
    return _merge(a + b, desc)


def _topk_kernel(x_ref, o_ref):
    x = x_ref[0]
    left = [x[i * G0:i * G0 + G0 // 2] for i in range(K)]
    right = [x[i * G0 + G0 // 2:(i + 1) * G0] for i in range(K)]
    cur_d = _sort(left, True)
    cur_a = _sort(right, False)
    g = G0 // 2
    while True:
        m = [jnp.maximum(cur_d[i], cur_a[i]) for i in range(K)]
        if g == 1:
            o_ref[0] = jnp.concatenate(_merge(m, True), axis=0)
            return
        h = g // 2
        cur_d = _merge([p[:h] for p in m], True)
        cur_a = _merge([p[h:] for p in m], False)
        g = h


def kernel(inputs):
    B, s, C = inputs.shape
    assert s == S and C % BC == 0
    grid = (B, C // BC)
    return pl.pallas_call(
        _topk_kernel,
        grid=grid,
        in_specs=[pl.BlockSpec((1, S, BC), lambda b, c: (b, 0, c))],
        out_specs=pl.BlockSpec((1, K, BC), lambda b, c: (b, 0, c)),
        out_shape=jax.ShapeDtypeStruct((B, K, C), jnp.float32),
    )(inputs)
